# baseline (speedup 1.0000x reference)
import functools

import jax
import jax.numpy as jnp
from jax import lax
from jax.experimental import pallas as pl
from jax.experimental.pallas import tpu as pltpu
from jax.experimental.pallas import tpu_sc as plsc

N_ = 10000
E_ = 320000
H_ = 128
G_ = 16
OUT_ = 10
NC_ = 2
NS_ = 16
NW_ = NC_ * NS_
K_ = 40
CH_ = E_ // (NW_ * K_)
CH2_ = E_ // (NS_ * K_)
HW_ = H_ // NC_
NFLY_ = 5
NPAIR_ = CH2_ // (2 * NFLY_)
ROWS_T_ = N_ // NS_
RCH_ = 125
DW_ = 16
BN_ = 2000
NB_ = N_ // BN_


def _sc_mesh():
    return plsc.VectorSubcoreMesh(
        core_axis_name="c", subcore_axis_name="s",
        num_cores=NC_, num_subcores=NS_)


def _sc_spmv(fsplit, src2, dst2):
    @functools.partial(
        pl.kernel,
        out_type=jax.ShapeDtypeStruct((N_, H_), jnp.float32),
        mesh=_sc_mesh(),
        scratch_types=[
            pltpu.VMEM((CH2_, K_), jnp.int32),
            pltpu.VMEM((CH2_, K_), jnp.int32),
            pltpu.VMEM((NFLY_, K_, HW_), jnp.float32),
            pltpu.VMEM((NFLY_, K_, HW_), jnp.float32),
            pltpu.VMEM((RCH_, HW_), jnp.float32),
            pltpu.VMEM_SHARED((N_, HW_), jnp.float32),
            pltpu.SemaphoreType.DMA,
            pltpu.SemaphoreType.DMA,
        ],
        compiler_params=pltpu.CompilerParams(use_tc_tiling_on_sc=False),
    )
    def run(f_hbm, src_hbm, dst_hbm, out_hbm, idx_s, idx_d, gbufa, gbufb,
            obuf, acc, semg, sems):
        c = lax.axis_index("c")
        s = lax.axis_index("s")
        pltpu.sync_copy(src_hbm.at[s], idx_s)
        pltpu.sync_copy(dst_hbm.at[s], idx_d)

        zero16 = jnp.zeros((16,), jnp.float32)

        def zrow(i, carry):
            for j in range(HW_ // 16):
                obuf[i, pl.ds(j * 16, 16)] = zero16
            return carry

        lax.fori_loop(0, RCH_, zrow, 0)
        base = s * ROWS_T_
        for z in range(ROWS_T_ // RCH_):
            pltpu.sync_copy(obuf, acc.at[pl.ds(base + z * RCH_, RCH_)])
        plsc.subcore_barrier()

        ftab = f_hbm.at[c]

        def fire_g(p, g):
            for j in range(NFLY_):
                pltpu.async_copy(ftab.at[idx_s.at[g * NFLY_ + j]],
                                 (gbufa if p == 0 else gbufb).at[j], semg)

        def wait_g(p):
            for j in range(NFLY_):
                pltpu.make_async_copy(ftab.at[idx_s.at[0]], (gbufa if p == 0 else gbufb).at[j],
                                      semg).wait()

        def fire_s(p, g):
            for j in range(NFLY_):
                pltpu.async_copy((gbufa if p == 0 else gbufb).at[j],
                                 acc.at[idx_d.at[g * NFLY_ + j]], sems,
                                 add=True)

        def wait_s(p):
            for j in range(NFLY_):
                pltpu.make_async_copy((gbufa if p == 0 else gbufb).at[j],
                                      acc.at[idx_d.at[0]], sems).wait()

        fire_g(0, 0)
        wait_g(0)
        fire_g(1, 1)
        fire_s(0, 0)
        wait_g(1)
        wait_s(0)
        fire_g(0, 2)
        fire_s(1, 1)

        def pair(i, carry):
            wait_g(0)
            wait_s(1)
            fire_g(1, 2 * i + 1)
            fire_s(0, 2 * i)
            wait_g(1)
            wait_s(0)
            fire_g(0, 2 * i + 2)
            fire_s(1, 2 * i + 1)
            return carry

        lax.fori_loop(1, NPAIR_ - 1, pair, 0)
        wait_g(0)
        wait_s(1)
        fire_g(1, 2 * (NPAIR_ - 1) + 1)
        fire_s(0, 2 * (NPAIR_ - 1))
        wait_g(1)
        wait_s(0)
        fire_s(1, 2 * (NPAIR_ - 1) + 1)
        wait_s(1)
        plsc.subcore_barrier()

        for z in range(ROWS_T_ // RCH_):
            r0 = base + z * RCH_
            pltpu.sync_copy(acc.at[pl.ds(r0, RCH_)], obuf)
            pltpu.sync_copy(obuf, out_hbm.at[pl.ds(r0, RCH_),
                                             pl.ds(c * HW_, HW_)])

    return run(fsplit, src2, dst2)


def _sc_deg(dst3):
    @functools.partial(
        pl.kernel,
        out_type=jax.ShapeDtypeStruct((NC_, N_, DW_), jnp.float32),
        mesh=_sc_mesh(),
        scratch_types=[
            pltpu.VMEM((CH_, K_), jnp.int32),
            pltpu.VMEM((K_, DW_), jnp.float32),
            pltpu.VMEM((RCH_, DW_), jnp.float32),
            pltpu.VMEM_SHARED((N_, DW_), jnp.float32),
        ],
        compiler_params=pltpu.CompilerParams(use_tc_tiling_on_sc=False),
    )
    def run(dst_hbm, out_hbm, idx_d, ones_b, obuf, accd):
        c = lax.axis_index("c")
        s = lax.axis_index("s")
        wid = s * NC_ + c
        pltpu.sync_copy(dst_hbm.at[wid], idx_d)

        one16 = jnp.ones((16,), jnp.float32)
        zero16 = jnp.zeros((16,), jnp.float32)

        def orow(i, carry):
            ones_b[i, :] = one16
            return carry

        lax.fori_loop(0, K_, orow, 0)

        def zrow(i, carry):
            obuf[i, :] = zero16
            return carry

        lax.fori_loop(0, RCH_, zrow, 0)
        base = s * ROWS_T_
        for z in range(ROWS_T_ // RCH_):
            pltpu.sync_copy(obuf, accd.at[pl.ds(base + z * RCH_, RCH_)])
        plsc.subcore_barrier()

        def chunk(i, carry):
            pltpu.sync_copy(ones_b, accd.at[idx_d.at[i]], add=True)
            return carry

        lax.fori_loop(0, CH_, chunk, 0)
        plsc.subcore_barrier()

        for z in range(ROWS_T_ // RCH_):
            r0 = base + z * RCH_
            pltpu.sync_copy(accd.at[pl.ds(r0, RCH_)], obuf)
            pltpu.sync_copy(obuf, out_hbm.at[c, pl.ds(r0, RCH_)])

    return run(dst3)


def _ln(v):
    m = jnp.mean(v, axis=-1, keepdims=True)
    d = v - m
    var = jnp.mean(d * d, axis=-1, keepdims=True)
    return d * lax.rsqrt(var + 1e-5)


def _relu(v):
    return jnp.maximum(v, 0.0)


def _dot(a, b):
    return jnp.dot(a, b, preferred_element_type=jnp.float32)


def _mean_of(S_ref, degp_ref):
    deg = jnp.maximum(degp_ref[0, :, 0:1] + degp_ref[1, :, 0:1], 1.0)
    return S_ref[...] / deg


def _split(fs_ref, v):
    fs_ref[0] = v[:, 0:HW_]
    fs_ref[1] = v[:, HW_:H_]


def _fuse1(t, ff, Wc_ref):
    return ((ff[0, 0] + ff[0, 1] + ff[0, 2]) * _relu(t)
            + ff[0, 3] * _relu(_dot(t, Wc_ref[...])))


def _fuse2(a0, a1, ff, Wc_ref):
    ssum = a0 + a1
    smax = jnp.maximum(a0, a1)
    cat = _dot(a0, Wc_ref[0:H_, :]) + _dot(a1, Wc_ref[H_:2 * H_, :])
    return (ff[1, 0] * _relu(ssum) + ff[1, 1] * _relu(ssum * 0.5)
            + ff[1, 2] * _relu(smax) + ff[1, 3] * _relu(cat))


def _fuse3(a0, a1, a2, ff, Wc_ref):
    ssum = a0 + a1 + a2
    smax = jnp.maximum(jnp.maximum(a0, a1), a2)
    cat = (_dot(a0, Wc_ref[0:H_, :]) + _dot(a1, Wc_ref[H_:2 * H_, :])
           + _dot(a2, Wc_ref[2 * H_:3 * H_, :]))
    return (ff[2, 0] * _relu(ssum) + ff[2, 1] * _relu(ssum / 3.0)
            + ff[2, 2] * _relu(smax) + ff[2, 3] * _relu(cat))


_ROW = pl.BlockSpec((BN_, H_), lambda i: (i, 0))
_FSP = pl.BlockSpec((NC_, BN_, HW_), lambda i: (0, i, 0))
_DEGP = pl.BlockSpec((NC_, BN_, DW_), lambda i: (0, i, 0))
_W = pl.BlockSpec((H_, H_), lambda i: (0, 0))
_W2 = pl.BlockSpec((2 * H_, H_), lambda i: (0, 0))
_W3 = pl.BlockSpec((3 * H_, H_), lambda i: (0, 0))
_B = pl.BlockSpec((1, H_), lambda i: (0, 0))
_SMEM = pl.BlockSpec(memory_space=pltpu.SMEM)


_FS_SHAPE = jax.ShapeDtypeStruct((NC_, N_, HW_), jnp.float32)
_ROW_SHAPE = jax.ShapeDtypeStruct((N_, H_), jnp.float32)


def _joined(fs_ref):
    return jnp.concatenate([fs_ref[0], fs_ref[1]], axis=-1)


def _tck1(sc_w, ff_w, x, W1, b1, Wc0):
    def body(sc, ff, x_ref, W1_ref, b1_ref, Wc0_ref, h_ref, f0_ref):
        h = _dot(x_ref[...], W1_ref[...]) + b1_ref[...]
        t = sc[0, 1] * h
        h_ref[...] = h
        _split(f0_ref, _fuse1(t, ff, Wc0_ref))

    return pl.pallas_call(
        body,
        grid=(NB_,),
        in_specs=[_SMEM, _SMEM, _ROW, _W, _B, _W],
        out_specs=[_ROW, _FSP],
        out_shape=[_ROW_SHAPE, _FS_SHAPE],
    )(sc_w, ff_w, x, W1, b1, Wc0)


def _tck2(sc_w, ff_w, f0, h, S0p, degp, Wl, Wr, b, Wc1):
    def body(sc, ff, f0_ref, h_ref, S0_ref, dg_ref, Wl_ref, Wr_ref, b_ref,
             Wc1_ref, hh_ref, f1_ref):
        mean = _mean_of(S0_ref, dg_ref)
        hh = _dot(_joined(f0_ref), Wl_ref[...]) + _dot(mean, Wr_ref[...]) \
            + b_ref[...]
        hh = _ln(_relu(hh))
        a0 = sc[1, 1] * h_ref[...]
        a1 = sc[2, 1] * hh
        hh_ref[...] = hh
        _split(f1_ref, _fuse2(a0, a1, ff, Wc1_ref))

    return pl.pallas_call(
        body,
        grid=(NB_,),
        in_specs=[_SMEM, _SMEM, _FSP, _ROW, _ROW, _DEGP, _W, _W, _B, _W2],
        out_specs=[_ROW, _FSP],
        out_shape=[_ROW_SHAPE, _FS_SHAPE],
    )(sc_w, ff_w, f0, h, S0p, degp, Wl, Wr, b, Wc1)


def _tck3(sc_w, ff_w, f1, S1p, degp, h, hh1, Wl, Wr, b, Wc2, c0W, c0b, Wc0):
    def body(sc, ff, f1_ref, S1_ref, dg_ref, h_ref, hh1_ref, Wl_ref, Wr_ref,
             b_ref, Wc2_ref, c0W_ref, c0b_ref, Wc0_ref, s1_ref, f2_ref):
        mean = _mean_of(S1_ref, dg_ref)
        hh2 = _dot(_joined(f1_ref), Wl_ref[...]) + _dot(mean, Wr_ref[...]) \
            + b_ref[...]
        hh2 = _ln(_relu(hh2))
        a0 = sc[3, 1] * h_ref[...]
        a1 = sc[4, 1] * hh1_ref[...]
        a2 = sc[5, 1] * hh2
        fo = _fuse3(a0, a1, a2, ff, Wc2_ref)
        s1 = _dot(fo, c0W_ref[...]) + c0b_ref[...]
        s1_ref[...] = s1
        t = sc[0, 1] * s1
        _split(f2_ref, _fuse1(t, ff, Wc0_ref))

    return pl.pallas_call(
        body,
        grid=(NB_,),
        in_specs=[_SMEM, _SMEM, _FSP, _ROW, _DEGP, _ROW, _ROW, _W, _W, _B,
                  _W3, _W, _B, _W],
        out_specs=[_ROW, _FSP],
        out_shape=[_ROW_SHAPE, _FS_SHAPE],
    )(sc_w, ff_w, f1, S1p, degp, h, hh1, Wl, Wr, b, Wc2, c0W, c0b, Wc0)


def _tck4(sc_w, ff_w, f2, S2p, degp, s1, Wl, Wr, b, Wc1):
    def body(sc, ff, f2_ref, S2_ref, dg_ref, s1_ref, Wl_ref, Wr_ref, b_ref,
             Wc1_ref, hh3_ref, f3_ref):
        mean = _mean_of(S2_ref, dg_ref)
        hh3 = _dot(_joined(f2_ref), Wl_ref[...]) + _dot(mean, Wr_ref[...]) \
            + b_ref[...]
        hh3 = _ln(_relu(hh3))
        a0 = sc[1, 1] * s1_ref[...]
        a1 = sc[2, 1] * hh3
        hh3_ref[...] = hh3
        _split(f3_ref, _fuse2(a0, a1, ff, Wc1_ref))

    return pl.pallas_call(
        body,
        grid=(NB_,),
        in_specs=[_SMEM, _SMEM, _FSP, _ROW, _DEGP, _ROW, _W, _W, _B, _W2],
        out_specs=[_ROW, _FSP],
        out_shape=[_ROW_SHAPE, _FS_SHAPE],
    )(sc_w, ff_w, f2, S2p, degp, s1, Wl, Wr, b, Wc1)


def _tck5(sc_w, ff_w, rw, batch, f3, S3p, degp, s1, hh3, Wl, Wr, b, Wc2,
          c1W, c1b, roW, rob, clfW, clfb):
    def body(sc, ff, rw_ref, bat_ref, f3_ref, S3_ref, dg_ref, s1_ref, hh3_ref,
             Wl_ref, Wr_ref, b_ref, Wc2_ref, c1W_ref, c1b_ref, roW_ref,
             rob_ref, clfW_ref, clfb_ref, out_ref, gsum, gmax, cnt):
        pid = pl.program_id(0)

        @pl.when(pid == 0)
        def _init():
            gsum[...] = jnp.zeros((G_, H_), jnp.float32)
            cnt[...] = jnp.zeros((G_, H_), jnp.float32)
            gmax[...] = jnp.full((G_, H_), -3.4e38, jnp.float32)

        mean = _mean_of(S3_ref, dg_ref)
        hh4 = _dot(_joined(f3_ref), Wl_ref[...]) + _dot(mean, Wr_ref[...]) \
            + b_ref[...]
        hh4 = _ln(_relu(hh4))
        a0 = sc[3, 1] * s1_ref[...]
        a1 = sc[4, 1] * hh3_ref[...]
        a2 = sc[5, 1] * hh4
        fo = _fuse3(a0, a1, a2, ff, Wc2_ref)
        s2 = _dot(fo, c1W_ref[...]) + c1b_ref[...]

        bat = bat_ref[...]
        ohf = (bat == lax.broadcasted_iota(jnp.int32, (BN_, G_), 1)
               .astype(jnp.float32)).astype(jnp.float32)
        gsum[...] += lax.dot_general(ohf, s2, (((0,), (0,)), ((), ())),
                                     preferred_element_type=jnp.float32)
        colsum = jnp.sum(ohf, axis=0)
        cnt[...] += jnp.broadcast_to(colsum.reshape(G_, 1), (G_, H_))
        for g in range(G_):
            row = jnp.max(jnp.where(bat == float(g), s2, -3.4e38), axis=0)
            gmax[g, :] = jnp.maximum(gmax[g, :], row)

        gmean = gsum[...] / jnp.maximum(cnt[...], 1.0)
        gmix = rw_ref[0] * gmean + rw_ref[1] * gmax[...] + rw_ref[2] * gsum[...]
        gr = _relu(_dot(gmix, roW_ref[...]) + rob_ref[...])
        out_ref[...] = _dot(gr, clfW_ref[...]) + clfb_ref[...]

    return pl.pallas_call(
        body,
        grid=(NB_,),
        in_specs=[_SMEM, _SMEM, _SMEM,
                  pl.BlockSpec((BN_, 1), lambda i: (i, 0)),
                  _FSP, _ROW, _DEGP, _ROW, _ROW, _W, _W, _B, _W3, _W, _B,
                  _W, pl.BlockSpec((1, H_), lambda i: (0, 0)),
                  pl.BlockSpec((H_, OUT_), lambda i: (0, 0)),
                  pl.BlockSpec((1, OUT_), lambda i: (0, 0))],
        out_specs=pl.BlockSpec((G_, OUT_), lambda i: (0, 0)),
        out_shape=jax.ShapeDtypeStruct((G_, OUT_), jnp.float32),
        scratch_shapes=[pltpu.VMEM((G_, H_), jnp.float32)] * 3,
    )(sc_w, ff_w, rw, batch, f3, S3p, degp, s1, hh3, Wl, Wr, b, Wc2, c1W,
      c1b, roW, rob, clfW, clfb)


def kernel(x, edge_index, batch, sc_w, ff_w, readout_w, params):
    p = params
    src3 = edge_index[0].reshape(NW_, CH_, K_)
    dst3 = edge_index[1].reshape(NW_, CH_, K_)
    src2 = edge_index[0].reshape(NS_, CH2_, K_)
    dst2 = edge_index[1].reshape(NS_, CH2_, K_)
    b1 = p['lin1_b'].reshape(1, H_)
    sb = [p['sage%d_b' % i].reshape(1, H_) for i in range(4)]
    c0b = p['cell0_b'].reshape(1, H_)
    c1b = p['cell1_b'].reshape(1, H_)
    rob = p['readout_b'].reshape(1, H_)
    clfb = p['clf_b'].reshape(1, OUT_)

    batc = batch.astype(jnp.float32).reshape(N_, 1)
    degp = _sc_deg(dst3)
    h, f0 = _tck1(sc_w, ff_w, x, p['lin1_W'], b1, p['ff_concat_0'])
    S0 = _sc_spmv(f0, src2, dst2)
    hh1, f1 = _tck2(sc_w, ff_w, f0, h, S0, degp, p['sage0_Wl'], p['sage0_Wr'],
                    sb[0], p['ff_concat_1'])
    S1 = _sc_spmv(f1, src2, dst2)
    s1, f2 = _tck3(sc_w, ff_w, f1, S1, degp, h, hh1, p['sage1_Wl'],
                   p['sage1_Wr'], sb[1], p['ff_concat_2'], p['cell0_W'], c0b,
                   p['ff_concat_0'])
    S2 = _sc_spmv(f2, src2, dst2)
    hh3, f3 = _tck4(sc_w, ff_w, f2, S2, degp, s1, p['sage2_Wl'],
                    p['sage2_Wr'], sb[2], p['ff_concat_1'])
    S3 = _sc_spmv(f3, src2, dst2)
    out = _tck5(sc_w, ff_w, readout_w, batc, f3, S3, degp, s1, hh3,
                p['sage3_Wl'], p['sage3_Wr'], sb[3], p['ff_concat_2'],
                p['cell1_W'], c1b, p['readout_W'], rob, p['clf_W'], clfb)
    return out


# speedup vs baseline: 1.0078x; 1.0078x over previous
import functools

import jax
import jax.numpy as jnp
from jax import lax
from jax.experimental import pallas as pl
from jax.experimental.pallas import tpu as pltpu
from jax.experimental.pallas import tpu_sc as plsc

N_ = 10000
E_ = 320000
H_ = 128
G_ = 16
OUT_ = 10
NC_ = 2
NS_ = 16
NW_ = NC_ * NS_
K_ = 40
CH_ = E_ // (NW_ * K_)
CH2_ = E_ // (NS_ * K_)
HW_ = H_ // NC_
NFLY_ = 5
NPAIR_ = CH2_ // (2 * NFLY_)
ROWS_T_ = N_ // NS_
RCH_ = 125
DW_ = 16
BN_ = 2000
NB_ = N_ // BN_


def _sc_mesh():
    return plsc.VectorSubcoreMesh(
        core_axis_name="c", subcore_axis_name="s",
        num_cores=NC_, num_subcores=NS_)


def _sc_spmv(fsplit, src2, dst2):
    @functools.partial(
        pl.kernel,
        out_type=jax.ShapeDtypeStruct((N_, H_), jnp.float32),
        mesh=_sc_mesh(),
        scratch_types=[
            pltpu.VMEM((CH2_, K_), jnp.int32),
            pltpu.VMEM((CH2_, K_), jnp.int32),
            pltpu.VMEM((NFLY_, K_, HW_), jnp.float32),
            pltpu.VMEM((NFLY_, K_, HW_), jnp.float32),
            pltpu.VMEM((RCH_, HW_), jnp.float32),
            pltpu.VMEM_SHARED((N_, HW_), jnp.float32),
            pltpu.SemaphoreType.DMA,
        ],
        compiler_params=pltpu.CompilerParams(use_tc_tiling_on_sc=False),
    )
    def run(f_hbm, src_hbm, dst_hbm, out_hbm, idx_s, idx_d, gbufa, gbufb,
            obuf, acc, semg):
        c = lax.axis_index("c")
        s = lax.axis_index("s")
        pltpu.sync_copy(src_hbm.at[s], idx_s)
        pltpu.sync_copy(dst_hbm.at[s], idx_d)

        zero16 = jnp.zeros((16,), jnp.float32)

        def zrow(i, carry):
            for j in range(HW_ // 16):
                obuf[i, pl.ds(j * 16, 16)] = zero16
            return carry

        lax.fori_loop(0, RCH_, zrow, 0)
        base = s * ROWS_T_
        for z in range(ROWS_T_ // RCH_):
            pltpu.sync_copy(obuf, acc.at[pl.ds(base + z * RCH_, RCH_)])
        plsc.subcore_barrier()

        ftab = f_hbm.at[c]

        def fire_g(p, g):
            for j in range(NFLY_):
                pltpu.async_copy(ftab.at[idx_s.at[g * NFLY_ + j]],
                                 (gbufa if p == 0 else gbufb).at[j], semg)

        def wait_g(p):
            for j in range(NFLY_):
                pltpu.make_async_copy(ftab.at[idx_s.at[0]], (gbufa if p == 0 else gbufb).at[j],
                                      semg).wait()

        def scat(p, g):
            for j in range(NFLY_):
                pltpu.sync_copy((gbufa if p == 0 else gbufb).at[j],
                                acc.at[idx_d.at[g * NFLY_ + j]], add=True)

        fire_g(0, 0)

        def pair(i, carry):
            wait_g(0)
            fire_g(1, 2 * i + 1)
            scat(0, 2 * i)
            wait_g(1)
            fire_g(0, 2 * i + 2)
            scat(1, 2 * i + 1)
            return carry

        lax.fori_loop(0, NPAIR_ - 1, pair, 0)
        wait_g(0)
        fire_g(1, 2 * (NPAIR_ - 1) + 1)
        scat(0, 2 * (NPAIR_ - 1))
        wait_g(1)
        scat(1, 2 * (NPAIR_ - 1) + 1)
        plsc.subcore_barrier()

        for z in range(ROWS_T_ // RCH_):
            r0 = base + z * RCH_
            pltpu.sync_copy(acc.at[pl.ds(r0, RCH_)], obuf)
            pltpu.sync_copy(obuf, out_hbm.at[pl.ds(r0, RCH_),
                                             pl.ds(c * HW_, HW_)])

    return run(fsplit, src2, dst2)


def _sc_deg(dst3):
    @functools.partial(
        pl.kernel,
        out_type=jax.ShapeDtypeStruct((NC_, N_, DW_), jnp.float32),
        mesh=_sc_mesh(),
        scratch_types=[
            pltpu.VMEM((CH_, K_), jnp.int32),
            pltpu.VMEM((K_, DW_), jnp.float32),
            pltpu.VMEM((RCH_, DW_), jnp.float32),
            pltpu.VMEM_SHARED((N_, DW_), jnp.float32),
        ],
        compiler_params=pltpu.CompilerParams(use_tc_tiling_on_sc=False),
    )
    def run(dst_hbm, out_hbm, idx_d, ones_b, obuf, accd):
        c = lax.axis_index("c")
        s = lax.axis_index("s")
        wid = s * NC_ + c
        pltpu.sync_copy(dst_hbm.at[wid], idx_d)

        one16 = jnp.ones((16,), jnp.float32)
        zero16 = jnp.zeros((16,), jnp.float32)

        def orow(i, carry):
            ones_b[i, :] = one16
            return carry

        lax.fori_loop(0, K_, orow, 0)

        def zrow(i, carry):
            obuf[i, :] = zero16
            return carry

        lax.fori_loop(0, RCH_, zrow, 0)
        base = s * ROWS_T_
        for z in range(ROWS_T_ // RCH_):
            pltpu.sync_copy(obuf, accd.at[pl.ds(base + z * RCH_, RCH_)])
        plsc.subcore_barrier()

        def chunk(i, carry):
            pltpu.sync_copy(ones_b, accd.at[idx_d.at[i]], add=True)
            return carry

        lax.fori_loop(0, CH_, chunk, 0)
        plsc.subcore_barrier()

        for z in range(ROWS_T_ // RCH_):
            r0 = base + z * RCH_
            pltpu.sync_copy(accd.at[pl.ds(r0, RCH_)], obuf)
            pltpu.sync_copy(obuf, out_hbm.at[c, pl.ds(r0, RCH_)])

    return run(dst3)


def _ln(v):
    m = jnp.mean(v, axis=-1, keepdims=True)
    d = v - m
    var = jnp.mean(d * d, axis=-1, keepdims=True)
    return d * lax.rsqrt(var + 1e-5)


def _relu(v):
    return jnp.maximum(v, 0.0)


def _dot(a, b):
    return jnp.dot(a, b, preferred_element_type=jnp.float32)


def _mean_of(S_ref, degp_ref):
    deg = jnp.maximum(degp_ref[0, :, 0:1] + degp_ref[1, :, 0:1], 1.0)
    return S_ref[...] / deg


def _split(fs_ref, v):
    fs_ref[0] = v[:, 0:HW_]
    fs_ref[1] = v[:, HW_:H_]


def _fuse1(t, ff, Wc_ref):
    return ((ff[0, 0] + ff[0, 1] + ff[0, 2]) * _relu(t)
            + ff[0, 3] * _relu(_dot(t, Wc_ref[...])))


def _fuse2(a0, a1, ff, Wc_ref):
    ssum = a0 + a1
    smax = jnp.maximum(a0, a1)
    cat = _dot(a0, Wc_ref[0:H_, :]) + _dot(a1, Wc_ref[H_:2 * H_, :])
    return (ff[1, 0] * _relu(ssum) + ff[1, 1] * _relu(ssum * 0.5)
            + ff[1, 2] * _relu(smax) + ff[1, 3] * _relu(cat))


def _fuse3(a0, a1, a2, ff, Wc_ref):
    ssum = a0 + a1 + a2
    smax = jnp.maximum(jnp.maximum(a0, a1), a2)
    cat = (_dot(a0, Wc_ref[0:H_, :]) + _dot(a1, Wc_ref[H_:2 * H_, :])
           + _dot(a2, Wc_ref[2 * H_:3 * H_, :]))
    return (ff[2, 0] * _relu(ssum) + ff[2, 1] * _relu(ssum / 3.0)
            + ff[2, 2] * _relu(smax) + ff[2, 3] * _relu(cat))


_ROW = pl.BlockSpec((BN_, H_), lambda i: (i, 0))
_FSP = pl.BlockSpec((NC_, BN_, HW_), lambda i: (0, i, 0))
_DEGP = pl.BlockSpec((NC_, BN_, DW_), lambda i: (0, i, 0))
_W = pl.BlockSpec((H_, H_), lambda i: (0, 0))
_W2 = pl.BlockSpec((2 * H_, H_), lambda i: (0, 0))
_W3 = pl.BlockSpec((3 * H_, H_), lambda i: (0, 0))
_B = pl.BlockSpec((1, H_), lambda i: (0, 0))
_SMEM = pl.BlockSpec(memory_space=pltpu.SMEM)


_FS_SHAPE = jax.ShapeDtypeStruct((NC_, N_, HW_), jnp.float32)
_ROW_SHAPE = jax.ShapeDtypeStruct((N_, H_), jnp.float32)


def _joined(fs_ref):
    return jnp.concatenate([fs_ref[0], fs_ref[1]], axis=-1)


def _tck1(sc_w, ff_w, x, W1, b1, Wc0):
    def body(sc, ff, x_ref, W1_ref, b1_ref, Wc0_ref, h_ref, f0_ref):
        h = _dot(x_ref[...], W1_ref[...]) + b1_ref[...]
        t = sc[0, 1] * h
        h_ref[...] = h
        _split(f0_ref, _fuse1(t, ff, Wc0_ref))

    return pl.pallas_call(
        body,
        grid=(NB_,),
        in_specs=[_SMEM, _SMEM, _ROW, _W, _B, _W],
        out_specs=[_ROW, _FSP],
        out_shape=[_ROW_SHAPE, _FS_SHAPE],
    )(sc_w, ff_w, x, W1, b1, Wc0)


def _tck2(sc_w, ff_w, f0, h, S0p, degp, Wl, Wr, b, Wc1):
    def body(sc, ff, f0_ref, h_ref, S0_ref, dg_ref, Wl_ref, Wr_ref, b_ref,
             Wc1_ref, hh_ref, f1_ref):
        mean = _mean_of(S0_ref, dg_ref)
        hh = _dot(_joined(f0_ref), Wl_ref[...]) + _dot(mean, Wr_ref[...]) \
            + b_ref[...]
        hh = _ln(_relu(hh))
        a0 = sc[1, 1] * h_ref[...]
        a1 = sc[2, 1] * hh
        hh_ref[...] = hh
        _split(f1_ref, _fuse2(a0, a1, ff, Wc1_ref))

    return pl.pallas_call(
        body,
        grid=(NB_,),
        in_specs=[_SMEM, _SMEM, _FSP, _ROW, _ROW, _DEGP, _W, _W, _B, _W2],
        out_specs=[_ROW, _FSP],
        out_shape=[_ROW_SHAPE, _FS_SHAPE],
    )(sc_w, ff_w, f0, h, S0p, degp, Wl, Wr, b, Wc1)


def _tck3(sc_w, ff_w, f1, S1p, degp, h, hh1, Wl, Wr, b, Wc2, c0W, c0b, Wc0):
    def body(sc, ff, f1_ref, S1_ref, dg_ref, h_ref, hh1_ref, Wl_ref, Wr_ref,
             b_ref, Wc2_ref, c0W_ref, c0b_ref, Wc0_ref, s1_ref, f2_ref):
        mean = _mean_of(S1_ref, dg_ref)
        hh2 = _dot(_joined(f1_ref), Wl_ref[...]) + _dot(mean, Wr_ref[...]) \
            + b_ref[...]
        hh2 = _ln(_relu(hh2))
        a0 = sc[3, 1] * h_ref[...]
        a1 = sc[4, 1] * hh1_ref[...]
        a2 = sc[5, 1] * hh2
        fo = _fuse3(a0, a1, a2, ff, Wc2_ref)
        s1 = _dot(fo, c0W_ref[...]) + c0b_ref[...]
        s1_ref[...] = s1
        t = sc[0, 1] * s1
        _split(f2_ref, _fuse1(t, ff, Wc0_ref))

    return pl.pallas_call(
        body,
        grid=(NB_,),
        in_specs=[_SMEM, _SMEM, _FSP, _ROW, _DEGP, _ROW, _ROW, _W, _W, _B,
                  _W3, _W, _B, _W],
        out_specs=[_ROW, _FSP],
        out_shape=[_ROW_SHAPE, _FS_SHAPE],
    )(sc_w, ff_w, f1, S1p, degp, h, hh1, Wl, Wr, b, Wc2, c0W, c0b, Wc0)


def _tck4(sc_w, ff_w, f2, S2p, degp, s1, Wl, Wr, b, Wc1):
    def body(sc, ff, f2_ref, S2_ref, dg_ref, s1_ref, Wl_ref, Wr_ref, b_ref,
             Wc1_ref, hh3_ref, f3_ref):
        mean = _mean_of(S2_ref, dg_ref)
        hh3 = _dot(_joined(f2_ref), Wl_ref[...]) + _dot(mean, Wr_ref[...]) \
            + b_ref[...]
        hh3 = _ln(_relu(hh3))
        a0 = sc[1, 1] * s1_ref[...]
        a1 = sc[2, 1] * hh3
        hh3_ref[...] = hh3
        _split(f3_ref, _fuse2(a0, a1, ff, Wc1_ref))

    return pl.pallas_call(
        body,
        grid=(NB_,),
        in_specs=[_SMEM, _SMEM, _FSP, _ROW, _DEGP, _ROW, _W, _W, _B, _W2],
        out_specs=[_ROW, _FSP],
        out_shape=[_ROW_SHAPE, _FS_SHAPE],
    )(sc_w, ff_w, f2, S2p, degp, s1, Wl, Wr, b, Wc1)


def _tck5(sc_w, ff_w, rw, batch, f3, S3p, degp, s1, hh3, Wl, Wr, b, Wc2,
          c1W, c1b, roW, rob, clfW, clfb):
    def body(sc, ff, rw_ref, bat_ref, f3_ref, S3_ref, dg_ref, s1_ref, hh3_ref,
             Wl_ref, Wr_ref, b_ref, Wc2_ref, c1W_ref, c1b_ref, roW_ref,
             rob_ref, clfW_ref, clfb_ref, out_ref, gsum, gmax, cnt):
        pid = pl.program_id(0)

        @pl.when(pid == 0)
        def _init():
            gsum[...] = jnp.zeros((G_, H_), jnp.float32)
            cnt[...] = jnp.zeros((G_, H_), jnp.float32)
            gmax[...] = jnp.full((G_, H_), -3.4e38, jnp.float32)

        mean = _mean_of(S3_ref, dg_ref)
        hh4 = _dot(_joined(f3_ref), Wl_ref[...]) + _dot(mean, Wr_ref[...]) \
            + b_ref[...]
        hh4 = _ln(_relu(hh4))
        a0 = sc[3, 1] * s1_ref[...]
        a1 = sc[4, 1] * hh3_ref[...]
        a2 = sc[5, 1] * hh4
        fo = _fuse3(a0, a1, a2, ff, Wc2_ref)
        s2 = _dot(fo, c1W_ref[...]) + c1b_ref[...]

        bat = bat_ref[...]
        ohf = (bat == lax.broadcasted_iota(jnp.int32, (BN_, G_), 1)
               .astype(jnp.float32)).astype(jnp.float32)
        gsum[...] += lax.dot_general(ohf, s2, (((0,), (0,)), ((), ())),
                                     preferred_element_type=jnp.float32)
        colsum = jnp.sum(ohf, axis=0)
        cnt[...] += jnp.broadcast_to(colsum.reshape(G_, 1), (G_, H_))
        for g in range(G_):
            row = jnp.max(jnp.where(bat == float(g), s2, -3.4e38), axis=0)
            gmax[g, :] = jnp.maximum(gmax[g, :], row)

        gmean = gsum[...] / jnp.maximum(cnt[...], 1.0)
        gmix = rw_ref[0] * gmean + rw_ref[1] * gmax[...] + rw_ref[2] * gsum[...]
        gr = _relu(_dot(gmix, roW_ref[...]) + rob_ref[...])
        out_ref[...] = _dot(gr, clfW_ref[...]) + clfb_ref[...]

    return pl.pallas_call(
        body,
        grid=(NB_,),
        in_specs=[_SMEM, _SMEM, _SMEM,
                  pl.BlockSpec((BN_, 1), lambda i: (i, 0)),
                  _FSP, _ROW, _DEGP, _ROW, _ROW, _W, _W, _B, _W3, _W, _B,
                  _W, pl.BlockSpec((1, H_), lambda i: (0, 0)),
                  pl.BlockSpec((H_, OUT_), lambda i: (0, 0)),
                  pl.BlockSpec((1, OUT_), lambda i: (0, 0))],
        out_specs=pl.BlockSpec((G_, OUT_), lambda i: (0, 0)),
        out_shape=jax.ShapeDtypeStruct((G_, OUT_), jnp.float32),
        scratch_shapes=[pltpu.VMEM((G_, H_), jnp.float32)] * 3,
    )(sc_w, ff_w, rw, batch, f3, S3p, degp, s1, hh3, Wl, Wr, b, Wc2, c1W,
      c1b, roW, rob, clfW, clfb)


def kernel(x, edge_index, batch, sc_w, ff_w, readout_w, params):
    p = params
    src3 = edge_index[0].reshape(NW_, CH_, K_)
    dst3 = edge_index[1].reshape(NW_, CH_, K_)
    src2 = edge_index[0].reshape(NS_, CH2_, K_)
    dst2 = edge_index[1].reshape(NS_, CH2_, K_)
    b1 = p['lin1_b'].reshape(1, H_)
    sb = [p['sage%d_b' % i].reshape(1, H_) for i in range(4)]
    c0b = p['cell0_b'].reshape(1, H_)
    c1b = p['cell1_b'].reshape(1, H_)
    rob = p['readout_b'].reshape(1, H_)
    clfb = p['clf_b'].reshape(1, OUT_)

    batc = batch.astype(jnp.float32).reshape(N_, 1)
    degp = _sc_deg(dst3)
    h, f0 = _tck1(sc_w, ff_w, x, p['lin1_W'], b1, p['ff_concat_0'])
    S0 = _sc_spmv(f0, src2, dst2)
    hh1, f1 = _tck2(sc_w, ff_w, f0, h, S0, degp, p['sage0_Wl'], p['sage0_Wr'],
                    sb[0], p['ff_concat_1'])
    S1 = _sc_spmv(f1, src2, dst2)
    s1, f2 = _tck3(sc_w, ff_w, f1, S1, degp, h, hh1, p['sage1_Wl'],
                   p['sage1_Wr'], sb[1], p['ff_concat_2'], p['cell0_W'], c0b,
                   p['ff_concat_0'])
    S2 = _sc_spmv(f2, src2, dst2)
    hh3, f3 = _tck4(sc_w, ff_w, f2, S2, degp, s1, p['sage2_Wl'],
                    p['sage2_Wr'], sb[2], p['ff_concat_1'])
    S3 = _sc_spmv(f3, src2, dst2)
    out = _tck5(sc_w, ff_w, readout_w, batc, f3, S3, degp, s1, hh3,
                p['sage3_Wl'], p['sage3_Wr'], sb[3], p['ff_concat_2'],
                p['cell1_W'], c1b, p['readout_W'], rob, p['clf_W'], clfb)
    return out
